# baseline (speedup 1.0000x reference)
import functools

import jax
import jax.numpy as jnp
from jax import lax
from jax.experimental import pallas as pl
from jax.experimental.pallas import tpu as pltpu
from jax.experimental.pallas import tpu_sc as plsc

B = 16384
EMB = 32
NU = 500000
NN = 2 * NU

_NC = 2
_NS = 16
_NW = _NC * _NS
_G = 2 * B
_BPW = _G // _NW


_CH = 128
_NCH = _BPW // _CH
_LTW = 16
_SEG = 2
_BPW2 = _BPW * _SEG
_NCH2 = _BPW2 // _CH


def _sc_gather_body(mem_hbm, lt_hbm, idx2_hbm, idx_hbm, hi_hbm, rows_out,
                    lt_out, idx2_v, idx_v, hi_v, rows_v, wide_v, lt_v,
                    sem_r, sem_l):
    wid = lax.axis_index("s") * _NC + lax.axis_index("c")
    base = wid * _BPW
    pltpu.sync_copy(idx2_hbm.at[wid], idx2_v)
    pltpu.sync_copy(idx_hbm.at[wid], idx_v)
    pltpu.sync_copy(hi_hbm.at[wid], hi_v)
    copies = []
    for j in range(_NCH2):
        copies.append(pltpu.async_copy(
            mem_hbm.at[idx2_v.at[j]], rows_v.at[pl.ds(j * _CH, _CH)], sem_r))
    for j in range(_NCH):
        copies.append(pltpu.async_copy(
            lt_hbm.at[hi_v.at[j]], wide_v.at[pl.ds(j * _CH, _CH)], sem_l))
    for c in copies:
        c.wait()
    lane = lax.iota(jnp.int32, 16)
    for k in range(_BPW // 16):
        c, o = divmod(k * 16, _CH)
        ids = idx_v[c, pl.ds(o, 16)]
        vals = plsc.load_gather(wide_v, [lane + k * 16,
                                         lax.bitwise_and(ids, _LTW - 1)])
        lt_v[pl.ds(k * 16, 16)] = vals
    pltpu.sync_copy(rows_v, rows_out.at[pl.ds(base * _SEG, _BPW2)])
    pltpu.sync_copy(lt_v, lt_out.at[pl.ds(base, _BPW)])


@functools.cache
def _sc_gather():
    return pl.kernel(
        _sc_gather_body,
        out_type=(jax.ShapeDtypeStruct((_G * _SEG, _LTW), jnp.float32),
                  jax.ShapeDtypeStruct((_G,), jnp.float32)),
        mesh=plsc.VectorSubcoreMesh(core_axis_name="c", subcore_axis_name="s"),
        scratch_types=[
            pltpu.VMEM((_NCH2, _CH), jnp.int32),
            pltpu.VMEM((_NCH, _CH), jnp.int32),
            pltpu.VMEM((_NCH, _CH), jnp.int32),
            pltpu.VMEM((_BPW2, _LTW), jnp.float32),
            pltpu.VMEM((_BPW, _LTW), jnp.float32),
            pltpu.VMEM((_BPW,), jnp.float32),
            pltpu.SemaphoreType.DMA,
            pltpu.SemaphoreType.DMA,
        ],
        compiler_params=pltpu.CompilerParams(use_tc_tiling_on_sc=False,
                                             needs_layout_passes=False),
    )


def _dense_body(ou_ref, oi_ref, lu_ref, li_ref, ts_ref, ft_ref, w_ref, b_ref,
                pr_ref, gu_ref, gi_ref):
    ou = ou_ref[...]
    oi = oi_ref[...]
    ft = ft_ref[...]
    ts = ts_ref[...]
    lu = lu_ref[...]
    li = li_ref[...]
    tw = b_ref[14]
    pu = ou * (1.0 + (ts - lu) * tw)
    pi = oi * (1.0 + (ts - li) * tw)

    def mm(x, k):
        return lax.dot_general(x, w_ref[k], (((1,), (0,)), ((), ())),
                               preferred_element_type=jnp.float32)

    def gru(xs, xp, h, wb, hb, bi, bh):
        gi_r = mm(xs, wb + 0) + mm(xp, wb + 1) + mm(ft, wb + 2) + b_ref[bi + 0]
        gi_z = mm(xs, wb + 3) + mm(xp, wb + 4) + mm(ft, wb + 5) + b_ref[bi + 1]
        gi_n = mm(xs, wb + 6) + mm(xp, wb + 7) + mm(ft, wb + 8) + b_ref[bi + 2]
        gh_r = mm(h, hb + 0) + b_ref[bh + 0]
        gh_z = mm(h, hb + 1) + b_ref[bh + 1]
        gh_n = mm(h, hb + 2) + b_ref[bh + 2]
        r = jax.nn.sigmoid(gi_r + gh_r)
        z = jax.nn.sigmoid(gi_z + gh_z)
        n = jnp.tanh(gi_n + r * gh_n)
        return (1.0 - z) * n + z * h

    nu_ = gru(pu, pi, ou, 0, 18, 0, 3)
    ni_ = gru(pi, pu, oi, 9, 21, 6, 9)
    g_u = jax.nn.sigmoid(mm(ou, 24) + mm(nu_, 25) + b_ref[12])
    gu = g_u * nu_ + (1.0 - g_u) * ou
    g_i = jax.nn.sigmoid(mm(oi, 24) + mm(ni_, 25) + b_ref[12])
    gi = g_i * ni_ + (1.0 - g_i) * oi
    pr_ref[...] = mm(gu, 26) + b_ref[13]
    gu_ref[...] = gu
    gi_ref[...] = gi


_BLK = 2048
_NB = B // _BLK


def _dense_specs():
    in_specs = [
        pl.BlockSpec((_BLK, EMB), lambda b: (b, 0)),
        pl.BlockSpec((_BLK, EMB), lambda b: (b + _NB, 0)),
        pl.BlockSpec((_BLK, 1), lambda b: (b, 0)),
        pl.BlockSpec((_BLK, 1), lambda b: (b + _NB, 0)),
        pl.BlockSpec((_BLK, 1), lambda b: (b, 0)),
        pl.BlockSpec((_BLK, EMB), lambda b: (b, 0)),
        pl.BlockSpec((27, 32, 32), lambda b: (0, 0, 0)),
        pl.BlockSpec((15, 1, 32), lambda b: (0, 0, 0)),
    ]
    out_specs = [pl.BlockSpec((_BLK, EMB), lambda b: (b, 0))] * 3
    return in_specs, out_specs


@functools.cache
def _dense_call():
    in_specs, out_specs = _dense_specs()
    return pl.pallas_call(
        _dense_body,
        grid=(_NB,),
        in_specs=in_specs,
        out_specs=out_specs,
        out_shape=[jax.ShapeDtypeStruct((B, EMB), jnp.float32)] * 3,
    )


def _pack_weights(time_w, u_Wih, u_Whh, u_bih, u_bhh, i_Wih, i_Whh, i_bih,
                  i_bhh, gate_w, gate_b, pred_w, pred_b):
    blocks = []
    for WT in (u_Wih.T, i_Wih.T):
        for g in range(3):
            for p in range(3):
                blocks.append(WT[32 * p:32 * (p + 1), 32 * g:32 * (g + 1)])
    for WhT in (u_Whh.T, i_Whh.T):
        for g in range(3):
            blocks.append(WhT[:, 32 * g:32 * (g + 1)])
    gwT = gate_w.T
    blocks.append(gwT[0:32])
    blocks.append(gwT[32:64])
    blocks.append(pred_w.T)
    wstk = jnp.stack(blocks)
    rows = []
    for bias in (u_bih, u_bhh, i_bih, i_bhh):
        for g in range(3):
            rows.append(bias[32 * g:32 * (g + 1)])
    rows.append(gate_b)
    rows.append(pred_b)
    rows.append(time_w[:, 0])
    bstk = jnp.stack(rows)[:, None, :]
    return wstk, bstk


def kernel(user_ids, item_ids, timestamps, features, query_time, memory,
           last_time, time_w, u_Wih, u_Whh, u_bih, u_bhh, i_Wih, i_Whh,
           i_bih, i_bhh, gate_w, gate_b, pred_w, pred_b):
    del query_time
    idx_all = jnp.concatenate([user_ids.astype(jnp.int32),
                               item_ids.astype(jnp.int32) + NU])
    idx3 = idx_all.reshape(_NW, _NCH, _CH)
    idx2 = (idx_all[:, None] * _SEG
            + jnp.arange(_SEG, dtype=jnp.int32)).reshape(_NW, _NCH2, _CH)
    rows16, lt_flat = _sc_gather()(memory.reshape(_SEG * NN, _LTW),
                                   last_time.reshape(NN // _LTW, _LTW),
                                   idx2, idx3, idx3 >> 4)
    rows_all = rows16.reshape(_G, EMB)
    lt_all = lt_flat[:, None]
    wstk, bstk = _pack_weights(time_w, u_Wih, u_Whh, u_bih, u_bhh, i_Wih,
                               i_Whh, i_bih, i_bhh, gate_w, gate_b, pred_w,
                               pred_b)
    pred, gated_u, gated_i = _dense_call()(
        rows_all, rows_all, lt_all, lt_all, timestamps[:, None], features,
        wstk, bstk)
    return (pred, gated_u, gated_i)


# speedup vs baseline: 1.4848x; 1.4848x over previous
import functools

import jax
import jax.numpy as jnp
from jax import lax
from jax.experimental import pallas as pl
from jax.experimental.pallas import tpu as pltpu
from jax.experimental.pallas import tpu_sc as plsc

B = 16384
EMB = 32
NU = 500000
NN = 2 * NU

_NC = 2
_NS = 16
_NW = _NC * _NS
_G = 2 * B
_BPW = _G // _NW


_CH = 128
_NCH = _BPW // _CH
_LTW = 16
_SEG = 2
_BPW2 = _BPW * _SEG
_NCH2 = _BPW2 // _CH


def _sc_gather_body(mem_hbm, lt_hbm, idx2_hbm, idx_hbm, hi_hbm, rows_out,
                    lt_out, idx2_v, idx_v, hi_v, rows_v, wide_v, lt_v,
                    sem_r, sem_l):
    wid = lax.axis_index("s") * _NC + lax.axis_index("c")
    base = wid * _BPW
    pltpu.sync_copy(idx2_hbm.at[wid], idx2_v)
    pltpu.sync_copy(idx_hbm.at[wid], idx_v)
    pltpu.sync_copy(hi_hbm.at[wid], hi_v)
    copies = []
    for j in range(_NCH2):
        copies.append(pltpu.async_copy(
            mem_hbm.at[idx2_v.at[j]], rows_v.at[pl.ds(j * _CH, _CH)], sem_r))
    for j in range(_NCH):
        copies.append(pltpu.async_copy(
            lt_hbm.at[hi_v.at[j]], wide_v.at[pl.ds(j * _CH, _CH)], sem_l))
    for c in copies:
        c.wait()
    lane = lax.iota(jnp.int32, 16)
    for k in range(_BPW // 16):
        c, o = divmod(k * 16, _CH)
        ids = idx_v[c, pl.ds(o, 16)]
        vals = plsc.load_gather(wide_v, [lane + k * 16,
                                         lax.bitwise_and(ids, _LTW - 1)])
        lt_v[pl.ds(k * 16, 16)] = vals
    pltpu.sync_copy(rows_v, rows_out.at[pl.ds(base * _SEG, _BPW2)])
    pltpu.sync_copy(lt_v, lt_out.at[pl.ds(base, _BPW)])


@functools.cache
def _sc_gather():
    return pl.kernel(
        _sc_gather_body,
        out_type=(jax.ShapeDtypeStruct((_G * _SEG, _LTW), jnp.float32),
                  jax.ShapeDtypeStruct((_G,), jnp.float32)),
        mesh=plsc.VectorSubcoreMesh(core_axis_name="c", subcore_axis_name="s"),
        scratch_types=[
            pltpu.VMEM((_NCH2, _CH), jnp.int32),
            pltpu.VMEM((_NCH, _CH), jnp.int32),
            pltpu.VMEM((_NCH, _CH), jnp.int32),
            pltpu.VMEM((_BPW2, _LTW), jnp.float32),
            pltpu.VMEM((_BPW, _LTW), jnp.float32),
            pltpu.VMEM((_BPW,), jnp.float32),
            pltpu.SemaphoreType.DMA,
            pltpu.SemaphoreType.DMA,
        ],
        compiler_params=pltpu.CompilerParams(use_tc_tiling_on_sc=False,
                                             needs_layout_passes=False),
    )


_CCH = 4096
_COM = _CCH * EMB // 1024
_CGRID = (NN + _CCH - 1) // _CCH


_CQ = _CCH // 4


def _conv_body(in_ref, out_ref):
    x = in_ref[...]
    eye = (lax.broadcasted_iota(jnp.int32, (EMB, EMB), 0) ==
           lax.broadcasted_iota(jnp.int32, (EMB, EMB), 1)).astype(jnp.float32)
    t = lax.dot_general(x, eye, (((0,), (0,)), ((), ())),
                        preferred_element_type=jnp.float32)
    out_ref[...] = jnp.concatenate(
        [t[a * _CQ:(a + 1) * _CQ, :] for a in range(4)], axis=1)


@functools.cache
def _conv_call():
    return pl.pallas_call(
        _conv_body,
        grid=(_CGRID,),
        in_specs=[pl.BlockSpec((EMB, _CCH), lambda b: (0, b))],
        out_specs=pl.BlockSpec((_CQ, 128), lambda b: (b, 0)),
        out_shape=jax.ShapeDtypeStruct((_CGRID * _CQ, 128), jnp.float32),
    )


def _dense_body(ou_ref, oi_ref, lu_ref, li_ref, ts_ref, ft_ref, w_ref, b_ref,
                pr_ref, gu_ref, gi_ref):
    ou = ou_ref[...]
    oi = oi_ref[...]
    ft = ft_ref[...]
    ts = ts_ref[...]
    lu = lu_ref[...]
    li = li_ref[...]
    tw = b_ref[14]
    pu = ou * (1.0 + (ts - lu) * tw)
    pi = oi * (1.0 + (ts - li) * tw)

    def mm(x, k):
        return lax.dot_general(x, w_ref[k], (((1,), (0,)), ((), ())),
                               preferred_element_type=jnp.float32)

    def gru(xs, xp, h, wb, hb, bi, bh):
        gi_r = mm(xs, wb + 0) + mm(xp, wb + 1) + mm(ft, wb + 2) + b_ref[bi + 0]
        gi_z = mm(xs, wb + 3) + mm(xp, wb + 4) + mm(ft, wb + 5) + b_ref[bi + 1]
        gi_n = mm(xs, wb + 6) + mm(xp, wb + 7) + mm(ft, wb + 8) + b_ref[bi + 2]
        gh_r = mm(h, hb + 0) + b_ref[bh + 0]
        gh_z = mm(h, hb + 1) + b_ref[bh + 1]
        gh_n = mm(h, hb + 2) + b_ref[bh + 2]
        r = jax.nn.sigmoid(gi_r + gh_r)
        z = jax.nn.sigmoid(gi_z + gh_z)
        n = jnp.tanh(gi_n + r * gh_n)
        return (1.0 - z) * n + z * h

    nu_ = gru(pu, pi, ou, 0, 18, 0, 3)
    ni_ = gru(pi, pu, oi, 9, 21, 6, 9)
    g_u = jax.nn.sigmoid(mm(ou, 24) + mm(nu_, 25) + b_ref[12])
    gu = g_u * nu_ + (1.0 - g_u) * ou
    g_i = jax.nn.sigmoid(mm(oi, 24) + mm(ni_, 25) + b_ref[12])
    gi = g_i * ni_ + (1.0 - g_i) * oi
    pr_ref[...] = mm(gu, 26) + b_ref[13]
    gu_ref[...] = gu
    gi_ref[...] = gi


_BLK = 2048
_NB = B // _BLK


def _dense_specs():
    in_specs = [
        pl.BlockSpec((_BLK, EMB), lambda b: (b, 0)),
        pl.BlockSpec((_BLK, EMB), lambda b: (b + _NB, 0)),
        pl.BlockSpec((_BLK, 1), lambda b: (b, 0)),
        pl.BlockSpec((_BLK, 1), lambda b: (b + _NB, 0)),
        pl.BlockSpec((_BLK, 1), lambda b: (b, 0)),
        pl.BlockSpec((_BLK, EMB), lambda b: (b, 0)),
        pl.BlockSpec((27, 32, 32), lambda b: (0, 0, 0)),
        pl.BlockSpec((15, 1, 32), lambda b: (0, 0, 0)),
    ]
    out_specs = [pl.BlockSpec((_BLK, EMB), lambda b: (b, 0))] * 3
    return in_specs, out_specs


@functools.cache
def _dense_call():
    in_specs, out_specs = _dense_specs()
    return pl.pallas_call(
        _dense_body,
        grid=(_NB,),
        in_specs=in_specs,
        out_specs=out_specs,
        out_shape=[jax.ShapeDtypeStruct((B, EMB), jnp.float32)] * 3,
    )


def _pack_weights(time_w, u_Wih, u_Whh, u_bih, u_bhh, i_Wih, i_Whh, i_bih,
                  i_bhh, gate_w, gate_b, pred_w, pred_b):
    blocks = []
    for WT in (u_Wih.T, i_Wih.T):
        for g in range(3):
            for p in range(3):
                blocks.append(WT[32 * p:32 * (p + 1), 32 * g:32 * (g + 1)])
    for WhT in (u_Whh.T, i_Whh.T):
        for g in range(3):
            blocks.append(WhT[:, 32 * g:32 * (g + 1)])
    gwT = gate_w.T
    blocks.append(gwT[0:32])
    blocks.append(gwT[32:64])
    blocks.append(pred_w.T)
    wstk = jnp.stack(blocks)
    rows = []
    for bias in (u_bih, u_bhh, i_bih, i_bhh):
        for g in range(3):
            rows.append(bias[32 * g:32 * (g + 1)])
    rows.append(gate_b)
    rows.append(pred_b)
    rows.append(time_w[:, 0])
    bstk = jnp.stack(rows)[:, None, :]
    return wstk, bstk


def kernel(user_ids, item_ids, timestamps, features, query_time, memory,
           last_time, time_w, u_Wih, u_Whh, u_bih, u_bhh, i_Wih, i_Whh,
           i_bih, i_bhh, gate_w, gate_b, pred_w, pred_b):
    del query_time
    idx_all = jnp.concatenate([user_ids.astype(jnp.int32),
                               item_ids.astype(jnp.int32) + NU])
    idx3 = idx_all.reshape(_NW, _NCH, _CH)
    rho = idx_all & (_CCH - 1)
    lrow = ((idx_all >> 12) << 10) | (rho & (_CQ - 1))
    g0 = lrow * 8 + (rho >> 10) * _SEG
    idx2 = (g0[:, None]
            + jnp.arange(_SEG, dtype=jnp.int32)).reshape(_NW, _NCH2, _CH)
    mem_lin = _conv_call()(memory.T).reshape(_CGRID * _CQ * 8, _LTW)
    rows16, lt_flat = _sc_gather()(mem_lin,
                                   last_time.reshape(NN // _LTW, _LTW),
                                   idx2, idx3, idx3 >> 4)
    rows_all = rows16.reshape(_G, EMB)
    lt_all = lt_flat[:, None]
    wstk, bstk = _pack_weights(time_w, u_Wih, u_Whh, u_bih, u_bhh, i_Wih,
                               i_Whh, i_bih, i_bhh, gate_w, gate_b, pred_w,
                               pred_b)
    pred, gated_u, gated_i = _dense_call()(
        rows_all, rows_all, lt_all, lt_all, timestamps[:, None], features,
        wstk, bstk)
    return (pred, gated_u, gated_i)
